# baseline (speedup 1.0000x reference)
import functools

import jax
import jax.numpy as jnp
from jax.experimental import pallas as pl


def _gate_tail(logits_ref, gates_ref, idx_ref):
    l = logits_ref[...]
    B, E = l.shape
    m1 = jnp.max(l, axis=1, keepdims=True)
    i1 = jnp.argmax(l, axis=1)
    masked = jnp.where(l >= m1, -1e30, l)
    m2 = jnp.max(masked, axis=1, keepdims=True)
    i2 = jnp.argmax(masked, axis=1)
    e2 = jnp.exp(m2 - m1)
    denom = 1.0 + e2
    g1 = 1.0 / denom
    g2 = e2 / denom
    lanes = jax.lax.broadcasted_iota(jnp.int32, (B, E), 1)
    gates = jnp.where(lanes == i1[:, None], g1, 0.0)
    gates = jnp.where(lanes == i2[:, None], g2, gates)
    gates_ref[...] = gates
    idx_ref[...] = jnp.stack([i1, i2], axis=1)


def _gate(logits):
    B, E = logits.shape
    return pl.pallas_call(
        _gate_tail,
        out_shape=(
            jax.ShapeDtypeStruct((B, E), jnp.float32),
            jax.ShapeDtypeStruct((B, 2), jnp.int32),
        ),
    )(logits)


def kernel(x, conv_w, conv_b, w1, b1, w2, b2):
    hi = jax.lax.Precision.HIGHEST
    y = jax.lax.conv_general_dilated(
        x, conv_w, window_strides=(4, 4), padding=[(3, 3), (3, 3)],
        dimension_numbers=("NCHW", "OIHW", "NCHW"), precision=hi)
    y = jax.nn.relu(y + conv_b[None, :, None, None])
    B, Cc, Hh, Ww = y.shape
    y = y.reshape(B, Cc, 4, Hh // 4, 4, Ww // 4).mean(axis=(3, 5))
    y = y.reshape(B, -1)
    y = jax.nn.relu(jnp.dot(y, w1.T, precision=hi) + b1)
    logits = jnp.dot(y, w2.T, precision=hi) + b2
    gates, idx = _gate(logits)
    return (gates, idx)


# speedup vs baseline: 2.8681x; 2.8681x over previous
import numpy as np

import jax
import jax.numpy as jnp
from jax.experimental import pallas as pl
from jax.experimental.pallas import tpu as pltpu

C = 96
OC = 32
HQ = 56
LW = 896
JP = 256
NKO = 224
CH = 14


def _sel_const():
    s = np.zeros((7 * JP, 64), np.float32)
    for kw in range(7):
        for wq in range(56):
            j = 4 * wq + kw - 3
            if 0 <= j < 224:
                s[kw * JP + j, wq] = 1.0
    return jnp.asarray(s, jnp.bfloat16)


def _pool_const():
    p = np.zeros((64, 128), np.float32)
    for wq in range(56):
        p[wq, wq // CH] = 1.0 / 196.0
    return jnp.asarray(p)


def _feature_body(xr_ref, wk_ref, ssel_ref, cbb_ref, pw_ref, w1g_ref,
                  w2t_ref, b1_ref, b2_ref, out_ref):
    X = xr_ref[0]
    xs_all = []
    for s in range(4):
        xs = X[:, :, 224 * s:224 * (s + 1)].astype(jnp.bfloat16)
        xs = jnp.concatenate(
            [xs, jnp.zeros((C, HQ, JP - 224), jnp.bfloat16)], axis=2)
        xs_all.append(xs)

    p4_rows = []
    for ph in range(4):
        T = None
        for kh in range(7):
            d = kh - 3
            s = d % 4
            ah = (d - s) // 4
            lo = CH * ph + ah
            xs = xs_all[s]
            if lo < 0:
                blk = jnp.concatenate(
                    [jnp.zeros((C, 1, JP), jnp.bfloat16), xs[:, 0:CH - 1, :]],
                    axis=1)
            else:
                blk = xs[:, lo:lo + CH, :]
            xsh = blk.reshape(C, CH * JP)
            t = jax.lax.dot_general(
                wk_ref[kh], xsh, (((0,), (0,)), ((), ())),
                preferred_element_type=jnp.float32)
            T = t if T is None else T + t
        T2 = T.reshape(NKO, CH, JP).reshape(NKO * CH, JP)
        Th = T2.astype(jnp.bfloat16)
        Tl = (T2 - Th.astype(jnp.float32)).astype(jnp.bfloat16)
        U = None
        for kw in range(7):
            sk = ssel_ref[kw * JP:(kw + 1) * JP, :]
            r0, r1 = kw * OC * CH, (kw + 1) * OC * CH
            u = (jax.lax.dot_general(
                    Th[r0:r1], sk, (((1,), (0,)), ((), ())),
                    preferred_element_type=jnp.float32)
                 + jax.lax.dot_general(
                    Tl[r0:r1], sk, (((1,), (0,)), ((), ())),
                    preferred_element_type=jnp.float32))
            U = u if U is None else U + u
        y = jnp.maximum(U + cbb_ref[...], 0.0)
        p4_rows.append(jnp.sum(y.reshape(OC, CH, 64), axis=1))

    p4 = jnp.concatenate(p4_rows, axis=0)
    p2 = jax.lax.dot_general(
        p4, pw_ref[...], (((1,), (0,)), ((), ())),
        preferred_element_type=jnp.float32)
    out4 = jax.lax.dot_general(
        p2, w1g_ref[...], (((0,), (0,)), ((), ())),
        preferred_element_type=jnp.float32)
    h1 = (out4[0:1, 0:64] + out4[1:2, 64:128]
          + out4[2:3, 128:192] + out4[3:4, 192:256])
    h1 = jnp.maximum(h1 + b1_ref[...], 0.0)
    logits = jax.lax.dot_general(
        h1, w2t_ref[...], (((1,), (0,)), ((), ())),
        preferred_element_type=jnp.float32) + b2_ref[...]
    out_ref[...] = logits[None]


def _gate_tail(logits_ref, gates_ref, idx_ref):
    l = logits_ref[...]
    B, E = l.shape
    m1 = jnp.max(l, axis=1, keepdims=True)
    i1 = jnp.argmax(l, axis=1)
    lanes = jax.lax.broadcasted_iota(jnp.int32, (B, E), 1)
    masked = jnp.where(lanes == i1[:, None], -jnp.inf, l)
    m2 = jnp.max(masked, axis=1, keepdims=True)
    i2 = jnp.argmax(masked, axis=1)
    e2 = jnp.exp(m2 - m1)
    denom = 1.0 + e2
    gates = jnp.where(lanes == i1[:, None], 1.0 / denom, 0.0)
    gates = jnp.where(lanes == i2[:, None], e2 / denom, gates)
    gates_ref[...] = gates
    idx_ref[...] = jnp.stack([i1, i2], axis=1)


def _full_spec(shape):
    nd = len(shape)
    return pl.BlockSpec(shape, lambda b, _n=nd: (0,) * _n)


def kernel(x, conv_w, conv_b, w1, b1, w2, b2):
    B = x.shape[0]
    xr = x.reshape(B, C, HQ, LW)
    wk = jnp.transpose(conv_w, (2, 1, 3, 0)).reshape(7, C, NKO)
    wk = wk.astype(jnp.bfloat16)
    ssel = _sel_const()
    cbb = jnp.broadcast_to(conv_b[:, None, None], (OC, CH, 64)).reshape(448, 64)
    pw = _pool_const()
    w1g = jnp.transpose(w1.reshape(64, OC, 4, 4), (2, 1, 3, 0)).reshape(128, 256)
    w2t = w2.T
    b1r = b1[None, :]
    b2r = b2[None, :]

    logits = pl.pallas_call(
        _feature_body,
        grid=(B,),
        in_specs=[
            pl.BlockSpec((1, C, HQ, LW), lambda b: (b, 0, 0, 0)),
            _full_spec((7, C, NKO)),
            _full_spec((7 * JP, 64)),
            _full_spec((448, 64)),
            _full_spec((64, 128)),
            _full_spec((128, 256)),
            _full_spec((64, 16)),
            _full_spec((1, 64)),
            _full_spec((1, 16)),
        ],
        out_specs=pl.BlockSpec((1, 1, 16), lambda b: (b, 0, 0)),
        out_shape=jax.ShapeDtypeStruct((B, 1, 16), jnp.float32),
        compiler_params=pltpu.CompilerParams(
            vmem_limit_bytes=63 * 1024 * 1024),
    )(xr, wk, ssel, cbb, pw, w1g, w2t, b1r, b2r)
    logits = logits.reshape(B, 16)

    gates, idx = pl.pallas_call(
        _gate_tail,
        out_shape=(
            jax.ShapeDtypeStruct((B, 16), jnp.float32),
            jax.ShapeDtypeStruct((B, 2), jnp.int32),
        ),
    )(logits)
    return (gates, idx)


# speedup vs baseline: 3.2994x; 1.1504x over previous
import numpy as np

import jax
import jax.numpy as jnp
from jax.experimental import pallas as pl
from jax.experimental.pallas import tpu as pltpu

C = 96
OC = 32
HQ = 56
LW = 896
JP = 256
NKO = 224
CH = 14


def _sel_const():
    s = np.zeros((7 * JP, 64), np.float32)
    for kw in range(7):
        for wq in range(56):
            j = 4 * wq + kw - 3
            if 0 <= j < 224:
                s[kw * JP + j, wq] = 1.0
    return jnp.asarray(s, jnp.bfloat16)


def _pool_const():
    p = np.zeros((64, 128), np.float32)
    for wq in range(56):
        p[wq, wq // CH] = 1.0 / 196.0
    return jnp.asarray(p)


def _feature_body(xr_ref, w0_ref, w1_ref, ssel_ref, cbb_ref, pw_ref, w1g_ref,
                  w2t_ref, b1_ref, b2_ref, out_ref):
    X = xr_ref[0]
    rows = []
    for s in range(4):
        xs = X[:, :, 224 * s:224 * (s + 1)].astype(jnp.bfloat16)
        xs = jax.lax.pad(xs, jnp.bfloat16(0),
                         ((0, 0, 0), (1, 0, 0), (0, JP - 224, 0)))
        rows.append(xs.reshape(C, (HQ + 1) * JP))
    xall = jnp.concatenate(rows, axis=0)

    p4_rows = []
    for ph in range(4):
        base = CH * JP * ph
        T = (jax.lax.dot_general(
                w0_ref[...], xall[:, base + JP:base + JP + CH * JP],
                (((0,), (0,)), ((), ())),
                preferred_element_type=jnp.float32)
             + jax.lax.dot_general(
                w1_ref[...], xall[96:, base:base + CH * JP],
                (((0,), (0,)), ((), ())),
                preferred_element_type=jnp.float32))
        T2 = T.reshape(NKO, CH, JP).reshape(NKO * CH, JP)
        Th = T2.astype(jnp.bfloat16)
        Tl = (T2 - Th.astype(jnp.float32)).astype(jnp.bfloat16)
        U = None
        for kw in range(7):
            sk = ssel_ref[kw * JP:(kw + 1) * JP, :]
            r0, r1 = kw * OC * CH, (kw + 1) * OC * CH
            u = (jax.lax.dot_general(
                    Th[r0:r1], sk, (((1,), (0,)), ((), ())),
                    preferred_element_type=jnp.float32)
                 + jax.lax.dot_general(
                    Tl[r0:r1], sk, (((1,), (0,)), ((), ())),
                    preferred_element_type=jnp.float32))
            U = u if U is None else U + u
        y = jnp.maximum(U + cbb_ref[...], 0.0)
        p4_rows.append(jnp.sum(y.reshape(OC, CH, 64), axis=1))

    p4 = jnp.concatenate(p4_rows, axis=0)
    p2 = jax.lax.dot_general(
        p4, pw_ref[...], (((1,), (0,)), ((), ())),
        preferred_element_type=jnp.float32)
    out4 = jax.lax.dot_general(
        p2, w1g_ref[...], (((0,), (0,)), ((), ())),
        preferred_element_type=jnp.float32)
    h1 = (out4[0:1, 0:64] + out4[1:2, 64:128]
          + out4[2:3, 128:192] + out4[3:4, 192:256])
    h1 = jnp.maximum(h1 + b1_ref[...], 0.0)
    logits = jax.lax.dot_general(
        h1, w2t_ref[...], (((1,), (0,)), ((), ())),
        preferred_element_type=jnp.float32) + b2_ref[...]
    out_ref[...] = logits[None]


def _gate_tail(logits_ref, gates_ref, idx_ref):
    l = logits_ref[...]
    B, E = l.shape
    m1 = jnp.max(l, axis=1, keepdims=True)
    i1 = jnp.argmax(l, axis=1)
    lanes = jax.lax.broadcasted_iota(jnp.int32, (B, E), 1)
    masked = jnp.where(lanes == i1[:, None], -jnp.inf, l)
    m2 = jnp.max(masked, axis=1, keepdims=True)
    i2 = jnp.argmax(masked, axis=1)
    e2 = jnp.exp(m2 - m1)
    denom = 1.0 + e2
    gates = jnp.where(lanes == i1[:, None], 1.0 / denom, 0.0)
    gates = jnp.where(lanes == i2[:, None], e2 / denom, gates)
    gates_ref[...] = gates
    idx_ref[...] = jnp.stack([i1, i2], axis=1)


def _full_spec(shape):
    nd = len(shape)
    return pl.BlockSpec(shape, lambda b, _n=nd: (0,) * _n)


def kernel(x, conv_w, conv_b, w1, b1, w2, b2):
    B = x.shape[0]
    xr = x.reshape(B, C, HQ, LW)
    wk = jnp.transpose(conv_w, (2, 1, 3, 0)).reshape(7, C, NKO)
    wk = wk.astype(jnp.bfloat16)
    w0p = wk[3:7].reshape(4 * C, NKO)
    w1p = wk[0:3].reshape(3 * C, NKO)
    ssel = _sel_const()
    cbb = jnp.broadcast_to(conv_b[:, None, None], (OC, CH, 64)).reshape(448, 64)
    pw = _pool_const()
    w1g = jnp.transpose(w1.reshape(64, OC, 4, 4), (2, 1, 3, 0)).reshape(128, 256)
    w2t = w2.T
    b1r = b1[None, :]
    b2r = b2[None, :]

    logits = pl.pallas_call(
        _feature_body,
        grid=(B,),
        in_specs=[
            pl.BlockSpec((1, C, HQ, LW), lambda b: (b, 0, 0, 0)),
            _full_spec((4 * C, NKO)),
            _full_spec((3 * C, NKO)),
            _full_spec((7 * JP, 64)),
            _full_spec((448, 64)),
            _full_spec((64, 128)),
            _full_spec((128, 256)),
            _full_spec((64, 16)),
            _full_spec((1, 64)),
            _full_spec((1, 16)),
        ],
        out_specs=pl.BlockSpec((1, 1, 16), lambda b: (b, 0, 0)),
        out_shape=jax.ShapeDtypeStruct((B, 1, 16), jnp.float32),
        compiler_params=pltpu.CompilerParams(
            vmem_limit_bytes=63 * 1024 * 1024),
    )(xr, w0p, w1p, ssel, cbb, pw, w1g, w2t, b1r, b2r)
    logits = logits.reshape(B, 16)

    gates, idx = pl.pallas_call(
        _gate_tail,
        out_shape=(
            jax.ShapeDtypeStruct((B, 16), jnp.float32),
            jax.ShapeDtypeStruct((B, 2), jnp.int32),
        ),
    )(logits)
    return (gates, idx)
